# Initial kernel scaffold; baseline (speedup 1.0000x reference)
import jax
import jax.numpy as jnp
from jax.experimental import pallas as pl


def kernel(x, edge_index, W1, b1, W2, b2, Wd, bd):
    raise NotImplementedError("write your pallas kernel here")


# speedup vs baseline: 13.6443x; 13.6443x over previous
import functools

import jax
import jax.numpy as jnp
from jax import lax
from jax.experimental import pallas as pl
from jax.experimental.pallas import tpu as pltpu
from jax.experimental.pallas import tpu_sc as plsc

N = 10000
R = 10240
E = 320000
CHUNK = 128
NW = 32
NCHUNK = -(-E // (NW * CHUNK))
EPW = NCHUNK * CHUNK
E_PAD = EPW * NW
RPT = R // 16

_mesh = lambda: plsc.VectorSubcoreMesh(core_axis_name="c", subcore_axis_name="s")


def _make_seg_sum(d):
    @functools.partial(
        pl.kernel,
        out_type=jax.ShapeDtypeStruct((2, R, d), jnp.float32),
        mesh=_mesh(),
        scratch_types=[
            pltpu.VMEM((CHUNK,), jnp.int32),
            pltpu.VMEM((CHUNK,), jnp.int32),
            pltpu.VMEM((CHUNK, d), jnp.float32),
            pltpu.VMEM_SHARED((R, d), jnp.float32),
            pltpu.SemaphoreType.DMA,
        ],
        compiler_params=pltpu.CompilerParams(use_tc_tiling_on_sc=False),
    )
    def seg_sum(g_hbm, src_hbm, dst_hbm, zeros_hbm, out_hbm,
                src_v, dst_v, msg_v, acc, sem):
        c = lax.axis_index("c")
        s = lax.axis_index("s")
        row0 = s * RPT
        pltpu.sync_copy(zeros_hbm.at[pl.ds(row0, RPT)], acc.at[pl.ds(row0, RPT)])
        plsc.subcore_barrier()
        base = (c * 16 + s) * EPW

        def body(i, carry):
            off = base + i * CHUNK
            pltpu.sync_copy(src_hbm.at[pl.ds(off, CHUNK)], src_v)
            pltpu.sync_copy(dst_hbm.at[pl.ds(off, CHUNK)], dst_v)
            pltpu.async_copy(g_hbm.at[src_v], msg_v, sem).wait()
            pltpu.sync_copy(msg_v, acc.at[dst_v], add=True)
            return carry

        lax.fori_loop(0, NCHUNK, body, 0)
        plsc.subcore_barrier()
        pltpu.sync_copy(acc.at[pl.ds(row0, RPT)],
                        out_hbm.at[c, pl.ds(row0, RPT)])

    return seg_sum


_seg64 = _make_seg_sum(64)
_seg32 = _make_seg_sum(32)


@functools.partial(
    pl.kernel,
    out_type=jax.ShapeDtypeStruct((2, R, 8), jnp.float32),
    mesh=_mesh(),
    scratch_types=[
        pltpu.VMEM((CHUNK,), jnp.int32),
        pltpu.VMEM((CHUNK, 8), jnp.float32),
        pltpu.VMEM_SHARED((R, 8), jnp.float32),
    ],
    compiler_params=pltpu.CompilerParams(use_tc_tiling_on_sc=False),
)
def _degree(dst_hbm, zeros_hbm, ones_hbm, out_hbm, dst_v, ones_v, acc):
    c = lax.axis_index("c")
    s = lax.axis_index("s")
    row0 = s * RPT
    pltpu.sync_copy(zeros_hbm.at[pl.ds(row0, RPT)], acc.at[pl.ds(row0, RPT)])
    pltpu.sync_copy(ones_hbm, ones_v)
    plsc.subcore_barrier()
    base = (c * 16 + s) * EPW

    def body(i, carry):
        off = base + i * CHUNK
        pltpu.sync_copy(dst_hbm.at[pl.ds(off, CHUNK)], dst_v)
        pltpu.sync_copy(ones_v, acc.at[dst_v], add=True)
        return carry

    lax.fori_loop(0, NCHUNK, body, 0)
    plsc.subcore_barrier()
    pltpu.sync_copy(acc.at[pl.ds(row0, RPT)], out_hbm.at[c, pl.ds(row0, RPT)])


def _dinv_of(dp_ref):
    deg = dp_ref[0, :, 0:1] + dp_ref[1, :, 0:1] + 1.0
    return lax.rsqrt(deg)


def _tc1_body(x_ref, w1_ref, dp_ref, g1_ref):
    t1 = jnp.dot(x_ref[...], w1_ref[...], preferred_element_type=jnp.float32)
    g1_ref[...] = t1 * _dinv_of(dp_ref)


def _tc2_body(a_ref, g1_ref, dp_ref, b1_ref, w2_ref, g2_ref):
    dinv = _dinv_of(dp_ref)
    h = jnp.maximum(dinv * (a_ref[0] + a_ref[1] + g1_ref[...]) + b1_ref[...], 0.0)
    t2 = jnp.dot(h, w2_ref[...], preferred_element_type=jnp.float32)
    g2_ref[...] = t2 * dinv


def _tc3_body(a_ref, g2_ref, dp_ref, b2_ref, wd_ref, bd_ref, xh_ref):
    dinv = _dinv_of(dp_ref)
    z = dinv * (a_ref[0] + a_ref[1] + g2_ref[...]) + b2_ref[...]
    xh_ref[...] = (jnp.dot(z, wd_ref[...], preferred_element_type=jnp.float32)
                   + bd_ref[...])


_BLK = 128
_G = R // _BLK


def _tc1(x_pad, W1, degp):
    return pl.pallas_call(
        _tc1_body,
        grid=(_G,),
        in_specs=[
            pl.BlockSpec((_BLK, 128), lambda i: (i, 0)),
            pl.BlockSpec((128, 64), lambda i: (0, 0)),
            pl.BlockSpec((2, _BLK, 8), lambda i: (0, i, 0)),
        ],
        out_specs=pl.BlockSpec((_BLK, 64), lambda i: (i, 0)),
        out_shape=jax.ShapeDtypeStruct((R, 64), jnp.float32),
    )(x_pad, W1, degp)


def _tc2(acc1, g1, degp, b1, W2):
    return pl.pallas_call(
        _tc2_body,
        grid=(_G,),
        in_specs=[
            pl.BlockSpec((2, _BLK, 64), lambda i: (0, i, 0)),
            pl.BlockSpec((_BLK, 64), lambda i: (i, 0)),
            pl.BlockSpec((2, _BLK, 8), lambda i: (0, i, 0)),
            pl.BlockSpec((1, 64), lambda i: (0, 0)),
            pl.BlockSpec((64, 32), lambda i: (0, 0)),
        ],
        out_specs=pl.BlockSpec((_BLK, 32), lambda i: (i, 0)),
        out_shape=jax.ShapeDtypeStruct((R, 32), jnp.float32),
    )(acc1, g1, degp, b1, W2)


def _tc3(acc2, g2, degp, b2, Wd, bd):
    return pl.pallas_call(
        _tc3_body,
        grid=(_G,),
        in_specs=[
            pl.BlockSpec((2, _BLK, 32), lambda i: (0, i, 0)),
            pl.BlockSpec((_BLK, 32), lambda i: (i, 0)),
            pl.BlockSpec((2, _BLK, 8), lambda i: (0, i, 0)),
            pl.BlockSpec((1, 32), lambda i: (0, 0)),
            pl.BlockSpec((32, 128), lambda i: (0, 0)),
            pl.BlockSpec((1, 128), lambda i: (0, 0)),
        ],
        out_specs=pl.BlockSpec((_BLK, 128), lambda i: (i, 0)),
        out_shape=jax.ShapeDtypeStruct((R, 128), jnp.float32),
    )(acc2, g2, degp, b2, Wd, bd)


def kernel(x, edge_index, W1, b1, W2, b2, Wd, bd):
    src = edge_index[0].astype(jnp.int32)
    dst = edge_index[1].astype(jnp.int32)
    pad = jnp.full((E_PAD - E,), N, jnp.int32)
    src_p = jnp.concatenate([src, pad])
    dst_p = jnp.concatenate([dst, pad])
    x_pad = jnp.pad(x, ((0, R - N), (0, 0)))

    zeros8 = jnp.zeros((R, 8), jnp.float32)
    zeros64 = jnp.zeros((R, 64), jnp.float32)
    zeros32 = jnp.zeros((R, 32), jnp.float32)
    ones8 = jnp.ones((CHUNK, 8), jnp.float32)

    degp = _degree(dst_p, zeros8, ones8)
    g1 = _tc1(x_pad, W1, degp)
    acc1 = _seg64(g1, src_p, dst_p, zeros64)
    g2 = _tc2(acc1, g1, degp, b1.reshape(1, 64), W2)
    acc2 = _seg32(g2, src_p, dst_p, zeros32)
    xh = _tc3(acc2, g2, degp, b2.reshape(1, 32), Wd, bd.reshape(1, 128))
    return xh[:N]


# speedup vs baseline: 16.1937x; 1.1868x over previous
import functools

import jax
import jax.numpy as jnp
from jax import lax
from jax.experimental import pallas as pl
from jax.experimental.pallas import tpu as pltpu
from jax.experimental.pallas import tpu_sc as plsc

N = 10000
R = 10240
E = 320000
CHUNK = 128
NW = 32
NCHUNK = 80
E_PAD = NW * NCHUNK * CHUNK
RPT = R // 16
NB = 4
NG = NCHUNK // NB

_mesh = lambda: plsc.VectorSubcoreMesh(core_axis_name="c", subcore_axis_name="s")
_sc_params = lambda: pltpu.CompilerParams(use_tc_tiling_on_sc=False)


def _make_seg_sum(d):
    @functools.partial(
        pl.kernel,
        out_type=jax.ShapeDtypeStruct((2, R, d), jnp.float32),
        mesh=_mesh(),
        scratch_types=[
            pltpu.VMEM((NCHUNK, CHUNK), jnp.int32),
            pltpu.VMEM((NCHUNK, CHUNK), jnp.int32),
            pltpu.VMEM((NB, CHUNK, d), jnp.float32),
            pltpu.VMEM_SHARED((R, d), jnp.float32),
            pltpu.SemaphoreType.DMA((NB,)),
            pltpu.SemaphoreType.DMA((NB,)),
        ],
        compiler_params=_sc_params(),
    )
    def seg_sum(g_hbm, src_hbm, dst_hbm, zeros_hbm, out_hbm,
                src_v, dst_v, msg, acc, sem_g, sem_s):
        c = lax.axis_index("c")
        s = lax.axis_index("s")
        w = c * 16 + s
        row0 = s * RPT
        pltpu.sync_copy(zeros_hbm.at[pl.ds(row0, RPT)], acc.at[pl.ds(row0, RPT)])
        pltpu.sync_copy(src_hbm.at[w], src_v)
        pltpu.sync_copy(dst_hbm.at[w], dst_v)
        plsc.subcore_barrier()

        def gather(j, b):
            pltpu.async_copy(g_hbm.at[src_v.at[j]], msg.at[b], sem_g.at[b])

        def gather_wait(j, b):
            pltpu.make_async_copy(g_hbm.at[src_v.at[j]], msg.at[b],
                                  sem_g.at[b]).wait()

        def scatter(j, b):
            pltpu.async_copy(msg.at[b], acc.at[dst_v.at[j]], sem_s.at[b],
                             add=True)

        def scatter_wait(j, b):
            pltpu.make_async_copy(msg.at[b], acc.at[dst_v.at[j]],
                                  sem_s.at[b]).wait()

        for b in range(NB):
            gather(b, b)

        def group(gi, carry):
            for b in range(NB):
                j = gi * NB + b
                gather_wait(j, b)
                scatter(j, b)
            for b in range(NB):
                j = gi * NB + b
                scatter_wait(j, b)
                gather(j + NB, b)
            return carry

        lax.fori_loop(0, NG - 1, group, 0)
        for b in range(NB):
            j = (NG - 1) * NB + b
            gather_wait(j, b)
            scatter(j, b)
        for b in range(NB):
            j = (NG - 1) * NB + b
            scatter_wait(j, b)

        plsc.subcore_barrier()
        pltpu.sync_copy(acc.at[pl.ds(row0, RPT)],
                        out_hbm.at[c, pl.ds(row0, RPT)])

    return seg_sum


_seg64 = _make_seg_sum(64)
_seg32 = _make_seg_sum(32)


@functools.partial(
    pl.kernel,
    out_type=jax.ShapeDtypeStruct((2, R, 8), jnp.float32),
    mesh=_mesh(),
    scratch_types=[
        pltpu.VMEM((NCHUNK, CHUNK), jnp.int32),
        pltpu.VMEM((CHUNK, 8), jnp.float32),
        pltpu.VMEM_SHARED((R, 8), jnp.float32),
        pltpu.SemaphoreType.DMA,
    ],
    compiler_params=_sc_params(),
)
def _degree(dst_hbm, zeros_hbm, ones_hbm, out_hbm, dst_v, ones_v, acc, sem):
    c = lax.axis_index("c")
    s = lax.axis_index("s")
    w = c * 16 + s
    row0 = s * RPT
    pltpu.sync_copy(zeros_hbm.at[pl.ds(row0, RPT)], acc.at[pl.ds(row0, RPT)])
    pltpu.sync_copy(ones_hbm, ones_v)
    pltpu.sync_copy(dst_hbm.at[w], dst_v)
    plsc.subcore_barrier()

    def fire(j, carry):
        pltpu.async_copy(ones_v, acc.at[dst_v.at[j]], sem, add=True)
        return carry

    lax.fori_loop(0, NCHUNK, fire, 0)

    def drain(j, carry):
        pltpu.make_async_copy(ones_v, acc.at[dst_v.at[j]], sem).wait()
        return carry

    lax.fori_loop(0, NCHUNK, drain, 0)
    plsc.subcore_barrier()
    pltpu.sync_copy(acc.at[pl.ds(row0, RPT)], out_hbm.at[c, pl.ds(row0, RPT)])


def _dinv_of(dp_ref):
    deg = dp_ref[0, :, 0:1] + dp_ref[1, :, 0:1] + 1.0
    return lax.rsqrt(deg)


def _tc1_body(x_ref, w1_ref, dp_ref, g1_ref):
    t1 = jnp.dot(x_ref[...], w1_ref[...], preferred_element_type=jnp.float32)
    g1_ref[...] = t1 * _dinv_of(dp_ref)


def _tc2_body(a_ref, g1_ref, dp_ref, b1_ref, w2_ref, g2_ref):
    dinv = _dinv_of(dp_ref)
    h = jnp.maximum(dinv * (a_ref[0] + a_ref[1] + g1_ref[...]) + b1_ref[...], 0.0)
    t2 = jnp.dot(h, w2_ref[...], preferred_element_type=jnp.float32)
    g2_ref[...] = t2 * dinv


def _tc3_body(a_ref, g2_ref, dp_ref, b2_ref, wd_ref, bd_ref, xh_ref):
    dinv = _dinv_of(dp_ref)
    z = dinv * (a_ref[0] + a_ref[1] + g2_ref[...]) + b2_ref[...]
    xh_ref[...] = (jnp.dot(z, wd_ref[...], preferred_element_type=jnp.float32)
                   + bd_ref[...])


_BLK = 128
_G = R // _BLK


def _tc1(x_pad, W1, degp):
    return pl.pallas_call(
        _tc1_body,
        grid=(_G,),
        in_specs=[
            pl.BlockSpec((_BLK, 128), lambda i: (i, 0)),
            pl.BlockSpec((128, 64), lambda i: (0, 0)),
            pl.BlockSpec((2, _BLK, 8), lambda i: (0, i, 0)),
        ],
        out_specs=pl.BlockSpec((_BLK, 64), lambda i: (i, 0)),
        out_shape=jax.ShapeDtypeStruct((R, 64), jnp.float32),
    )(x_pad, W1, degp)


def _tc2(acc1, g1, degp, b1, W2):
    return pl.pallas_call(
        _tc2_body,
        grid=(_G,),
        in_specs=[
            pl.BlockSpec((2, _BLK, 64), lambda i: (0, i, 0)),
            pl.BlockSpec((_BLK, 64), lambda i: (i, 0)),
            pl.BlockSpec((2, _BLK, 8), lambda i: (0, i, 0)),
            pl.BlockSpec((1, 64), lambda i: (0, 0)),
            pl.BlockSpec((64, 32), lambda i: (0, 0)),
        ],
        out_specs=pl.BlockSpec((_BLK, 32), lambda i: (i, 0)),
        out_shape=jax.ShapeDtypeStruct((R, 32), jnp.float32),
    )(acc1, g1, degp, b1, W2)


def _tc3(acc2, g2, degp, b2, Wd, bd):
    return pl.pallas_call(
        _tc3_body,
        grid=(_G,),
        in_specs=[
            pl.BlockSpec((2, _BLK, 32), lambda i: (0, i, 0)),
            pl.BlockSpec((_BLK, 32), lambda i: (i, 0)),
            pl.BlockSpec((2, _BLK, 8), lambda i: (0, i, 0)),
            pl.BlockSpec((1, 32), lambda i: (0, 0)),
            pl.BlockSpec((32, 128), lambda i: (0, 0)),
            pl.BlockSpec((1, 128), lambda i: (0, 0)),
        ],
        out_specs=pl.BlockSpec((_BLK, 128), lambda i: (i, 0)),
        out_shape=jax.ShapeDtypeStruct((R, 128), jnp.float32),
    )(acc2, g2, degp, b2, Wd, bd)


def kernel(x, edge_index, W1, b1, W2, b2, Wd, bd):
    src = edge_index[0].astype(jnp.int32)
    dst = edge_index[1].astype(jnp.int32)
    pad = jnp.full((E_PAD - E,), N, jnp.int32)
    src_p = jnp.concatenate([src, pad]).reshape(NW, NCHUNK, CHUNK)
    dst_p = jnp.concatenate([dst, pad]).reshape(NW, NCHUNK, CHUNK)
    x_pad = jnp.pad(x, ((0, R - N), (0, 0)))

    zeros8 = jnp.zeros((R, 8), jnp.float32)
    zeros64 = jnp.zeros((R, 64), jnp.float32)
    zeros32 = jnp.zeros((R, 32), jnp.float32)
    ones8 = jnp.ones((CHUNK, 8), jnp.float32)

    degp = _degree(dst_p, zeros8, ones8)
    g1 = _tc1(x_pad, W1, degp)
    acc1 = _seg64(g1, src_p, dst_p, zeros64)
    g2 = _tc2(acc1, g1, degp, b1.reshape(1, 64), W2)
    acc2 = _seg32(g2, src_p, dst_p, zeros32)
    xh = _tc3(acc2, g2, degp, b2.reshape(1, 32), Wd, bd.reshape(1, 128))
    return xh[:N]


# speedup vs baseline: 16.7457x; 1.0341x over previous
import functools

import jax
import jax.numpy as jnp
from jax import lax
from jax.experimental import pallas as pl
from jax.experimental.pallas import tpu as pltpu
from jax.experimental.pallas import tpu_sc as plsc

N = 10000
R = 10240
E = 320000
CHUNK = 128
NW = 32
NCHUNK = 80
E_PAD = NW * NCHUNK * CHUNK
RPT = R // 16
NB = 8
NG = NCHUNK // NB

_mesh = lambda: plsc.VectorSubcoreMesh(core_axis_name="c", subcore_axis_name="s")
_sc_params = lambda: pltpu.CompilerParams(use_tc_tiling_on_sc=False)


def _make_seg_sum(d):
    @functools.partial(
        pl.kernel,
        out_type=jax.ShapeDtypeStruct((2, R, d), jnp.float32),
        mesh=_mesh(),
        scratch_types=[
            pltpu.VMEM((NCHUNK, CHUNK), jnp.int32),
            pltpu.VMEM((NCHUNK, CHUNK), jnp.int32),
            pltpu.VMEM((NB, CHUNK, d), jnp.float32),
            pltpu.VMEM_SHARED((R, d), jnp.float32),
            pltpu.SemaphoreType.DMA((NB,)),
            pltpu.SemaphoreType.DMA((NB,)),
        ],
        compiler_params=_sc_params(),
    )
    def seg_sum(g_hbm, src_hbm, dst_hbm, zeros_hbm, out_hbm,
                src_v, dst_v, msg, acc, sem_g, sem_s):
        c = lax.axis_index("c")
        s = lax.axis_index("s")
        w = c * 16 + s
        row0 = s * RPT
        pltpu.sync_copy(zeros_hbm.at[pl.ds(row0, RPT)], acc.at[pl.ds(row0, RPT)])
        pltpu.sync_copy(src_hbm.at[w], src_v)
        pltpu.sync_copy(dst_hbm.at[w], dst_v)
        plsc.subcore_barrier()

        def gather(j, b):
            pltpu.async_copy(g_hbm.at[src_v.at[j]], msg.at[b], sem_g.at[b])

        def gather_wait(j, b):
            pltpu.make_async_copy(g_hbm.at[src_v.at[j]], msg.at[b],
                                  sem_g.at[b]).wait()

        def scatter(j, b):
            pltpu.async_copy(msg.at[b], acc.at[dst_v.at[j]], sem_s.at[b],
                             add=True)

        def scatter_wait(j, b):
            pltpu.make_async_copy(msg.at[b], acc.at[dst_v.at[j]],
                                  sem_s.at[b]).wait()

        for b in range(NB):
            gather(b, b)

        def group(gi, carry):
            for b in range(NB):
                j = gi * NB + b
                gather_wait(j, b)
                scatter(j, b)
            for b in range(NB):
                j = gi * NB + b
                scatter_wait(j, b)
                gather(j + NB, b)
            return carry

        lax.fori_loop(0, NG - 1, group, 0)
        for b in range(NB):
            j = (NG - 1) * NB + b
            gather_wait(j, b)
            scatter(j, b)
        for b in range(NB):
            j = (NG - 1) * NB + b
            scatter_wait(j, b)

        plsc.subcore_barrier()
        pltpu.sync_copy(acc.at[pl.ds(row0, RPT)],
                        out_hbm.at[c, pl.ds(row0, RPT)])

    return seg_sum


_seg64 = _make_seg_sum(64)
_seg32 = _make_seg_sum(32)


@functools.partial(
    pl.kernel,
    out_type=jax.ShapeDtypeStruct((2, R, 8), jnp.float32),
    mesh=_mesh(),
    scratch_types=[
        pltpu.VMEM((NCHUNK, CHUNK), jnp.int32),
        pltpu.VMEM((CHUNK, 8), jnp.float32),
        pltpu.VMEM_SHARED((R, 8), jnp.float32),
        pltpu.SemaphoreType.DMA,
    ],
    compiler_params=_sc_params(),
)
def _degree(dst_hbm, zeros_hbm, ones_hbm, out_hbm, dst_v, ones_v, acc, sem):
    c = lax.axis_index("c")
    s = lax.axis_index("s")
    w = c * 16 + s
    row0 = s * RPT
    pltpu.sync_copy(zeros_hbm.at[pl.ds(row0, RPT)], acc.at[pl.ds(row0, RPT)])
    pltpu.sync_copy(ones_hbm, ones_v)
    pltpu.sync_copy(dst_hbm.at[w], dst_v)
    plsc.subcore_barrier()

    def fire(j, carry):
        pltpu.async_copy(ones_v, acc.at[dst_v.at[j]], sem, add=True)
        return carry

    lax.fori_loop(0, NCHUNK, fire, 0)

    def drain(j, carry):
        pltpu.make_async_copy(ones_v, acc.at[dst_v.at[j]], sem).wait()
        return carry

    lax.fori_loop(0, NCHUNK, drain, 0)
    plsc.subcore_barrier()
    pltpu.sync_copy(acc.at[pl.ds(row0, RPT)], out_hbm.at[c, pl.ds(row0, RPT)])


def _dinv_of(dp_ref):
    deg = dp_ref[0, :, 0:1] + dp_ref[1, :, 0:1] + 1.0
    return lax.rsqrt(deg)


def _tc1_body(x_ref, w1_ref, dp_ref, g1_ref):
    t1 = jnp.dot(x_ref[...], w1_ref[...], preferred_element_type=jnp.float32)
    g1_ref[...] = t1 * _dinv_of(dp_ref)


def _tc2_body(a_ref, g1_ref, dp_ref, b1_ref, w2_ref, g2_ref):
    dinv = _dinv_of(dp_ref)
    h = jnp.maximum(dinv * (a_ref[0] + a_ref[1] + g1_ref[...]) + b1_ref[...], 0.0)
    t2 = jnp.dot(h, w2_ref[...], preferred_element_type=jnp.float32)
    g2_ref[...] = t2 * dinv


def _tc3_body(a_ref, g2_ref, dp_ref, b2_ref, wd_ref, bd_ref, xh_ref):
    dinv = _dinv_of(dp_ref)
    z = dinv * (a_ref[0] + a_ref[1] + g2_ref[...]) + b2_ref[...]
    xh_ref[...] = (jnp.dot(z, wd_ref[...], preferred_element_type=jnp.float32)
                   + bd_ref[...])


_BLK = 128
_G = R // _BLK


def _tc1(x_pad, W1, degp):
    return pl.pallas_call(
        _tc1_body,
        grid=(_G,),
        in_specs=[
            pl.BlockSpec((_BLK, 128), lambda i: (i, 0)),
            pl.BlockSpec((128, 64), lambda i: (0, 0)),
            pl.BlockSpec((2, _BLK, 8), lambda i: (0, i, 0)),
        ],
        out_specs=pl.BlockSpec((_BLK, 64), lambda i: (i, 0)),
        out_shape=jax.ShapeDtypeStruct((R, 64), jnp.float32),
    )(x_pad, W1, degp)


def _tc2(acc1, g1, degp, b1, W2):
    return pl.pallas_call(
        _tc2_body,
        grid=(_G,),
        in_specs=[
            pl.BlockSpec((2, _BLK, 64), lambda i: (0, i, 0)),
            pl.BlockSpec((_BLK, 64), lambda i: (i, 0)),
            pl.BlockSpec((2, _BLK, 8), lambda i: (0, i, 0)),
            pl.BlockSpec((1, 64), lambda i: (0, 0)),
            pl.BlockSpec((64, 32), lambda i: (0, 0)),
        ],
        out_specs=pl.BlockSpec((_BLK, 32), lambda i: (i, 0)),
        out_shape=jax.ShapeDtypeStruct((R, 32), jnp.float32),
    )(acc1, g1, degp, b1, W2)


def _tc3(acc2, g2, degp, b2, Wd, bd):
    return pl.pallas_call(
        _tc3_body,
        grid=(_G,),
        in_specs=[
            pl.BlockSpec((2, _BLK, 32), lambda i: (0, i, 0)),
            pl.BlockSpec((_BLK, 32), lambda i: (i, 0)),
            pl.BlockSpec((2, _BLK, 8), lambda i: (0, i, 0)),
            pl.BlockSpec((1, 32), lambda i: (0, 0)),
            pl.BlockSpec((32, 128), lambda i: (0, 0)),
            pl.BlockSpec((1, 128), lambda i: (0, 0)),
        ],
        out_specs=pl.BlockSpec((_BLK, 128), lambda i: (i, 0)),
        out_shape=jax.ShapeDtypeStruct((R, 128), jnp.float32),
    )(acc2, g2, degp, b2, Wd, bd)


def kernel(x, edge_index, W1, b1, W2, b2, Wd, bd):
    src = edge_index[0].astype(jnp.int32)
    dst = edge_index[1].astype(jnp.int32)
    pad = jnp.full((E_PAD - E,), N, jnp.int32)
    src_p = jnp.concatenate([src, pad]).reshape(NW, NCHUNK, CHUNK)
    dst_p = jnp.concatenate([dst, pad]).reshape(NW, NCHUNK, CHUNK)
    x_pad = jnp.pad(x, ((0, R - N), (0, 0)))

    zeros8 = jnp.zeros((R, 8), jnp.float32)
    zeros64 = jnp.zeros((R, 64), jnp.float32)
    zeros32 = jnp.zeros((R, 32), jnp.float32)
    ones8 = jnp.ones((CHUNK, 8), jnp.float32)

    degp = _degree(dst_p, zeros8, ones8)
    g1 = _tc1(x_pad, W1, degp)
    acc1 = _seg64(g1, src_p, dst_p, zeros64)
    g2 = _tc2(acc1, g1, degp, b1.reshape(1, 64), W2)
    acc2 = _seg32(g2, src_p, dst_p, zeros32)
    xh = _tc3(acc2, g2, degp, b2.reshape(1, 32), Wd, bd.reshape(1, 128))
    return xh[:N]


# speedup vs baseline: 24.7145x; 1.4759x over previous
import functools

import jax
import jax.numpy as jnp
from jax import lax
from jax.experimental import pallas as pl
from jax.experimental.pallas import tpu as pltpu
from jax.experimental.pallas import tpu_sc as plsc

N = 10000
R = 10240
E = 320000
CHUNK = 128
NW = 32
NCHUNK = 80
E_PAD = NW * NCHUNK * CHUNK
RPT = R // 16
NB = 8
NG = NCHUNK // NB

_mesh = lambda: plsc.VectorSubcoreMesh(core_axis_name="c", subcore_axis_name="s")
_sc_params = lambda: pltpu.CompilerParams(use_tc_tiling_on_sc=False)


def _make_seg_sum(d, stage_spmem):
    scratch = [
        pltpu.VMEM((NCHUNK, CHUNK), jnp.int32),
        pltpu.VMEM((NCHUNK, CHUNK), jnp.int32),
        pltpu.VMEM((NB, CHUNK, d), jnp.float32),
        pltpu.VMEM_SHARED((R, d), jnp.float32),
    ]
    if stage_spmem:
        scratch.append(pltpu.VMEM_SHARED((R, d), jnp.float32))
    scratch += [
        pltpu.SemaphoreType.DMA((NB,)),
        pltpu.SemaphoreType.DMA((NB,)),
    ]

    @functools.partial(
        pl.kernel,
        out_type=jax.ShapeDtypeStruct((2, R, d), jnp.float32),
        mesh=_mesh(),
        scratch_types=scratch,
        compiler_params=_sc_params(),
    )
    def seg_sum(g_hbm, src_hbm, dst_hbm, zeros_hbm, out_hbm,
                src_v, dst_v, msg, acc, *rest):
        if stage_spmem:
            g_sp, sem_g, sem_s = rest
        else:
            sem_g, sem_s = rest
            g_sp = None
        c = lax.axis_index("c")
        s = lax.axis_index("s")
        w = c * 16 + s
        row0 = s * RPT
        pltpu.sync_copy(zeros_hbm.at[pl.ds(row0, RPT)], acc.at[pl.ds(row0, RPT)])
        if stage_spmem:
            pltpu.sync_copy(g_hbm.at[pl.ds(row0, RPT)], g_sp.at[pl.ds(row0, RPT)])
        pltpu.sync_copy(src_hbm.at[w], src_v)
        pltpu.sync_copy(dst_hbm.at[w], dst_v)
        plsc.subcore_barrier()
        g_src = g_sp if stage_spmem else g_hbm

        def gather(j, b):
            pltpu.async_copy(g_src.at[src_v.at[j]], msg.at[b], sem_g.at[b])

        def gather_wait(j, b):
            pltpu.make_async_copy(g_src.at[src_v.at[j]], msg.at[b],
                                  sem_g.at[b]).wait()

        def scatter(j, b):
            pltpu.async_copy(msg.at[b], acc.at[dst_v.at[j]], sem_s.at[b],
                             add=True)

        def scatter_wait(j, b):
            pltpu.make_async_copy(msg.at[b], acc.at[dst_v.at[j]],
                                  sem_s.at[b]).wait()

        for b in range(NB):
            gather(b, b)

        def group(gi, carry):
            for b in range(NB):
                j = gi * NB + b
                gather_wait(j, b)
                scatter(j, b)
            for b in range(NB):
                j = gi * NB + b
                scatter_wait(j, b)
                gather(j + NB, b)
            return carry

        lax.fori_loop(0, NG - 1, group, 0)
        for b in range(NB):
            j = (NG - 1) * NB + b
            gather_wait(j, b)
            scatter(j, b)
        for b in range(NB):
            j = (NG - 1) * NB + b
            scatter_wait(j, b)

        plsc.subcore_barrier()
        pltpu.sync_copy(acc.at[pl.ds(row0, RPT)],
                        out_hbm.at[c, pl.ds(row0, RPT)])

    return seg_sum


_seg64 = _make_seg_sum(64, stage_spmem=False)
_seg32 = _make_seg_sum(32, stage_spmem=True)


@functools.partial(
    pl.kernel,
    out_type=jax.ShapeDtypeStruct((2, R, 8), jnp.float32),
    mesh=_mesh(),
    scratch_types=[
        pltpu.VMEM((NCHUNK, CHUNK), jnp.int32),
        pltpu.VMEM((CHUNK, 8), jnp.float32),
        pltpu.VMEM_SHARED((R, 8), jnp.float32),
        pltpu.SemaphoreType.DMA,
    ],
    compiler_params=_sc_params(),
)
def _degree(dst_hbm, zeros_hbm, ones_hbm, out_hbm, dst_v, ones_v, acc, sem):
    c = lax.axis_index("c")
    s = lax.axis_index("s")
    w = c * 16 + s
    row0 = s * RPT
    pltpu.sync_copy(zeros_hbm.at[pl.ds(row0, RPT)], acc.at[pl.ds(row0, RPT)])
    pltpu.sync_copy(ones_hbm, ones_v)
    pltpu.sync_copy(dst_hbm.at[w], dst_v)
    plsc.subcore_barrier()

    def fire(j, carry):
        pltpu.async_copy(ones_v, acc.at[dst_v.at[j]], sem, add=True)
        return carry

    lax.fori_loop(0, NCHUNK, fire, 0)

    def drain(j, carry):
        pltpu.make_async_copy(ones_v, acc.at[dst_v.at[j]], sem).wait()
        return carry

    lax.fori_loop(0, NCHUNK, drain, 0)
    plsc.subcore_barrier()
    pltpu.sync_copy(acc.at[pl.ds(row0, RPT)], out_hbm.at[c, pl.ds(row0, RPT)])


def _dinv_of(dp_ref):
    deg = dp_ref[0, :, 0:1] + dp_ref[1, :, 0:1] + 1.0
    return lax.rsqrt(deg)


def _tc1_body(x_ref, w1_ref, dp_ref, g1_ref):
    t1 = jnp.dot(x_ref[...], w1_ref[...], preferred_element_type=jnp.float32)
    g1_ref[...] = t1 * _dinv_of(dp_ref)


def _tc2_body(a_ref, g1_ref, dp_ref, b1_ref, w2_ref, g2_ref):
    dinv = _dinv_of(dp_ref)
    h = jnp.maximum(dinv * (a_ref[0] + a_ref[1] + g1_ref[...]) + b1_ref[...], 0.0)
    t2 = jnp.dot(h, w2_ref[...], preferred_element_type=jnp.float32)
    g2_ref[...] = t2 * dinv


def _tc3_body(a_ref, g2_ref, dp_ref, b2_ref, wd_ref, bd_ref, xh_ref):
    dinv = _dinv_of(dp_ref)
    z = dinv * (a_ref[0] + a_ref[1] + g2_ref[...]) + b2_ref[...]
    xh_ref[...] = (jnp.dot(z, wd_ref[...], preferred_element_type=jnp.float32)
                   + bd_ref[...])


_BLK = 1280
_G = R // _BLK


def _tc1(x_pad, W1, degp):
    return pl.pallas_call(
        _tc1_body,
        grid=(_G,),
        in_specs=[
            pl.BlockSpec((_BLK, 128), lambda i: (i, 0)),
            pl.BlockSpec((128, 64), lambda i: (0, 0)),
            pl.BlockSpec((2, _BLK, 8), lambda i: (0, i, 0)),
        ],
        out_specs=pl.BlockSpec((_BLK, 64), lambda i: (i, 0)),
        out_shape=jax.ShapeDtypeStruct((R, 64), jnp.float32),
    )(x_pad, W1, degp)


def _tc2(acc1, g1, degp, b1, W2):
    return pl.pallas_call(
        _tc2_body,
        grid=(_G,),
        in_specs=[
            pl.BlockSpec((2, _BLK, 64), lambda i: (0, i, 0)),
            pl.BlockSpec((_BLK, 64), lambda i: (i, 0)),
            pl.BlockSpec((2, _BLK, 8), lambda i: (0, i, 0)),
            pl.BlockSpec((1, 64), lambda i: (0, 0)),
            pl.BlockSpec((64, 32), lambda i: (0, 0)),
        ],
        out_specs=pl.BlockSpec((_BLK, 32), lambda i: (i, 0)),
        out_shape=jax.ShapeDtypeStruct((R, 32), jnp.float32),
    )(acc1, g1, degp, b1, W2)


def _tc3(acc2, g2, degp, b2, Wd, bd):
    return pl.pallas_call(
        _tc3_body,
        grid=(_G,),
        in_specs=[
            pl.BlockSpec((2, _BLK, 32), lambda i: (0, i, 0)),
            pl.BlockSpec((_BLK, 32), lambda i: (i, 0)),
            pl.BlockSpec((2, _BLK, 8), lambda i: (0, i, 0)),
            pl.BlockSpec((1, 32), lambda i: (0, 0)),
            pl.BlockSpec((32, 128), lambda i: (0, 0)),
            pl.BlockSpec((1, 128), lambda i: (0, 0)),
        ],
        out_specs=pl.BlockSpec((_BLK, 128), lambda i: (i, 0)),
        out_shape=jax.ShapeDtypeStruct((R, 128), jnp.float32),
    )(acc2, g2, degp, b2, Wd, bd)


def kernel(x, edge_index, W1, b1, W2, b2, Wd, bd):
    src = edge_index[0].astype(jnp.int32)
    dst = edge_index[1].astype(jnp.int32)
    pad = jnp.full((E_PAD - E,), N, jnp.int32)
    src_p = jnp.concatenate([src, pad]).reshape(NW, NCHUNK, CHUNK)
    dst_p = jnp.concatenate([dst, pad]).reshape(NW, NCHUNK, CHUNK)
    x_pad = jnp.pad(x, ((0, R - N), (0, 0)))

    zeros8 = jnp.zeros((R, 8), jnp.float32)
    zeros64 = jnp.zeros((R, 64), jnp.float32)
    zeros32 = jnp.zeros((R, 32), jnp.float32)
    ones8 = jnp.ones((CHUNK, 8), jnp.float32)

    degp = _degree(dst_p, zeros8, ones8)
    g1 = _tc1(x_pad, W1, degp)
    acc1 = _seg64(g1, src_p, dst_p, zeros64)
    g2 = _tc2(acc1, g1, degp, b1.reshape(1, 64), W2)
    acc2 = _seg32(g2, src_p, dst_p, zeros32)
    xh = _tc3(acc2, g2, degp, b2.reshape(1, 32), Wd, bd.reshape(1, 128))
    return xh[:N]


# speedup vs baseline: 36.6482x; 1.4829x over previous
import functools

import jax
import jax.numpy as jnp
from jax import lax
from jax.experimental import pallas as pl
from jax.experimental.pallas import tpu as pltpu
from jax.experimental.pallas import tpu_sc as plsc

N = 10000
R = 10240
E = 320000
CHUNK = 128
NCHUNK = 80
E_PAD = 32 * NCHUNK * CHUNK
NC2 = E_PAD // (16 * CHUNK)
RPT = R // 16
NB = 8
NG2 = NC2 // NB

_mesh = lambda: plsc.VectorSubcoreMesh(core_axis_name="c", subcore_axis_name="s")
_sc_params = lambda: pltpu.CompilerParams(use_tc_tiling_on_sc=False)


def _make_seg_half(dh):
    @functools.partial(
        pl.kernel,
        out_type=jax.ShapeDtypeStruct((2, R, dh), jnp.float32),
        mesh=_mesh(),
        scratch_types=[
            pltpu.VMEM((NC2, CHUNK), jnp.int32),
            pltpu.VMEM((NC2, CHUNK), jnp.int32),
            pltpu.VMEM((NB, CHUNK, dh), jnp.float32),
            pltpu.VMEM_SHARED((R, dh), jnp.float32),
            pltpu.VMEM_SHARED((R, dh), jnp.float32),
            pltpu.SemaphoreType.DMA((NB,)),
            pltpu.SemaphoreType.DMA((NB,)),
        ],
        compiler_params=_sc_params(),
    )
    def seg_half(g_hbm, src_hbm, dst_hbm, zeros_hbm, out_hbm,
                 src_v, dst_v, msg, acc, g_sp, sem_g, sem_s):
        c = lax.axis_index("c")
        s = lax.axis_index("s")
        row0 = s * RPT
        pltpu.sync_copy(zeros_hbm.at[pl.ds(row0, RPT)], acc.at[pl.ds(row0, RPT)])
        pltpu.sync_copy(g_hbm.at[c, pl.ds(row0, RPT)], g_sp.at[pl.ds(row0, RPT)])
        pltpu.sync_copy(src_hbm.at[s], src_v)
        pltpu.sync_copy(dst_hbm.at[s], dst_v)
        plsc.subcore_barrier()

        def gather(j, b):
            pltpu.async_copy(g_sp.at[src_v.at[j]], msg.at[b], sem_g.at[b])

        def gather_wait(j, b):
            pltpu.make_async_copy(g_sp.at[src_v.at[j]], msg.at[b],
                                  sem_g.at[b]).wait()

        def scatter(j, b):
            pltpu.async_copy(msg.at[b], acc.at[dst_v.at[j]], sem_s.at[b],
                             add=True)

        def scatter_wait(j, b):
            pltpu.make_async_copy(msg.at[b], acc.at[dst_v.at[j]],
                                  sem_s.at[b]).wait()

        for b in range(NB):
            gather(b, b)

        def group(gi, carry):
            for b in range(NB):
                j = gi * NB + b
                gather_wait(j, b)
                scatter(j, b)
            for b in range(NB):
                j = gi * NB + b
                scatter_wait(j, b)
                gather(j + NB, b)
            return carry

        lax.fori_loop(0, NG2 - 1, group, 0)
        for b in range(NB):
            j = (NG2 - 1) * NB + b
            gather_wait(j, b)
            scatter(j, b)
        for b in range(NB):
            j = (NG2 - 1) * NB + b
            scatter_wait(j, b)

        plsc.subcore_barrier()
        pltpu.sync_copy(acc.at[pl.ds(row0, RPT)],
                        out_hbm.at[c, pl.ds(row0, RPT)])

    return seg_half


_seg_l1 = _make_seg_half(32)
_seg_l2 = _make_seg_half(16)


@functools.partial(
    pl.kernel,
    out_type=jax.ShapeDtypeStruct((2, R, 8), jnp.float32),
    mesh=_mesh(),
    scratch_types=[
        pltpu.VMEM((NCHUNK, CHUNK), jnp.int32),
        pltpu.VMEM((CHUNK, 8), jnp.float32),
        pltpu.VMEM_SHARED((R, 8), jnp.float32),
        pltpu.SemaphoreType.DMA,
    ],
    compiler_params=_sc_params(),
)
def _degree(dst_hbm, zeros_hbm, ones_hbm, out_hbm, dst_v, ones_v, acc, sem):
    c = lax.axis_index("c")
    s = lax.axis_index("s")
    w = c * 16 + s
    row0 = s * RPT
    pltpu.sync_copy(zeros_hbm.at[pl.ds(row0, RPT)], acc.at[pl.ds(row0, RPT)])
    pltpu.sync_copy(ones_hbm, ones_v)
    pltpu.sync_copy(dst_hbm.at[w], dst_v)
    plsc.subcore_barrier()

    def fire(j, carry):
        pltpu.async_copy(ones_v, acc.at[dst_v.at[j]], sem, add=True)
        return carry

    lax.fori_loop(0, NCHUNK, fire, 0)

    def drain(j, carry):
        pltpu.make_async_copy(ones_v, acc.at[dst_v.at[j]], sem).wait()
        return carry

    lax.fori_loop(0, NCHUNK, drain, 0)
    plsc.subcore_barrier()
    pltpu.sync_copy(acc.at[pl.ds(row0, RPT)], out_hbm.at[c, pl.ds(row0, RPT)])


def _dinv_of(dp_ref):
    deg = dp_ref[0, :, 0:1] + dp_ref[1, :, 0:1] + 1.0
    return lax.rsqrt(deg)


def _cat(a_ref):
    return jnp.concatenate([a_ref[0], a_ref[1]], axis=-1)


def _tc1_body(x_ref, w1_ref, dp_ref, g1_ref):
    t1 = jnp.dot(x_ref[...], w1_ref[...], preferred_element_type=jnp.float32)
    g1 = t1 * _dinv_of(dp_ref)
    g1_ref[0] = g1[:, :32]
    g1_ref[1] = g1[:, 32:]


def _tc2_body(a_ref, g1_ref, dp_ref, b1_ref, w2_ref, g2_ref):
    dinv = _dinv_of(dp_ref)
    h = jnp.maximum(dinv * (_cat(a_ref) + _cat(g1_ref)) + b1_ref[...], 0.0)
    t2 = jnp.dot(h, w2_ref[...], preferred_element_type=jnp.float32)
    g2 = t2 * dinv
    g2_ref[0] = g2[:, :16]
    g2_ref[1] = g2[:, 16:]


def _tc3_body(a_ref, g2_ref, dp_ref, b2_ref, wd_ref, bd_ref, xh_ref):
    dinv = _dinv_of(dp_ref)
    z = dinv * (_cat(a_ref) + _cat(g2_ref)) + b2_ref[...]
    xh_ref[...] = (jnp.dot(z, wd_ref[...], preferred_element_type=jnp.float32)
                   + bd_ref[...])


_BLK = 1280
_G = R // _BLK


def _tc1(x_pad, W1, degp):
    return pl.pallas_call(
        _tc1_body,
        grid=(_G,),
        in_specs=[
            pl.BlockSpec((_BLK, 128), lambda i: (i, 0)),
            pl.BlockSpec((128, 64), lambda i: (0, 0)),
            pl.BlockSpec((2, _BLK, 8), lambda i: (0, i, 0)),
        ],
        out_specs=pl.BlockSpec((2, _BLK, 32), lambda i: (0, i, 0)),
        out_shape=jax.ShapeDtypeStruct((2, R, 32), jnp.float32),
    )(x_pad, W1, degp)


def _tc2(acc1, g1, degp, b1, W2):
    return pl.pallas_call(
        _tc2_body,
        grid=(_G,),
        in_specs=[
            pl.BlockSpec((2, _BLK, 32), lambda i: (0, i, 0)),
            pl.BlockSpec((2, _BLK, 32), lambda i: (0, i, 0)),
            pl.BlockSpec((2, _BLK, 8), lambda i: (0, i, 0)),
            pl.BlockSpec((1, 64), lambda i: (0, 0)),
            pl.BlockSpec((64, 32), lambda i: (0, 0)),
        ],
        out_specs=pl.BlockSpec((2, _BLK, 16), lambda i: (0, i, 0)),
        out_shape=jax.ShapeDtypeStruct((2, R, 16), jnp.float32),
    )(acc1, g1, degp, b1, W2)


_BLK3 = 1000
_G3 = N // _BLK3


def _tc3(acc2, g2, degp, b2, Wd, bd):
    return pl.pallas_call(
        _tc3_body,
        grid=(_G3,),
        in_specs=[
            pl.BlockSpec((2, _BLK3, 16), lambda i: (0, i, 0)),
            pl.BlockSpec((2, _BLK3, 16), lambda i: (0, i, 0)),
            pl.BlockSpec((2, _BLK3, 8), lambda i: (0, i, 0)),
            pl.BlockSpec((1, 32), lambda i: (0, 0)),
            pl.BlockSpec((32, 128), lambda i: (0, 0)),
            pl.BlockSpec((1, 128), lambda i: (0, 0)),
        ],
        out_specs=pl.BlockSpec((_BLK3, 128), lambda i: (i, 0)),
        out_shape=jax.ShapeDtypeStruct((N, 128), jnp.float32),
    )(acc2, g2, degp, b2, Wd, bd)


def kernel(x, edge_index, W1, b1, W2, b2, Wd, bd):
    src = edge_index[0].astype(jnp.int32)
    dst = edge_index[1].astype(jnp.int32)
    pad = jnp.full((E_PAD - E,), N, jnp.int32)
    src_p = jnp.concatenate([src, pad])
    dst_p = jnp.concatenate([dst, pad])
    src32 = src_p.reshape(32, NCHUNK, CHUNK)
    dst32 = dst_p.reshape(32, NCHUNK, CHUNK)
    src16 = src_p.reshape(16, NC2, CHUNK)
    dst16 = dst_p.reshape(16, NC2, CHUNK)
    x_pad = jnp.pad(x, ((0, R - N), (0, 0)))

    zeros8 = jnp.zeros((R, 8), jnp.float32)
    zeros32 = jnp.zeros((R, 32), jnp.float32)
    zeros16 = jnp.zeros((R, 16), jnp.float32)
    ones8 = jnp.ones((CHUNK, 8), jnp.float32)

    degp = _degree(dst32, zeros8, ones8)
    g1 = _tc1(x_pad, W1, degp)
    acc1 = _seg_l1(g1, src16, dst16, zeros32)
    g2 = _tc2(acc1, g1, degp, b1.reshape(1, 64), W2)
    acc2 = _seg_l2(g2, src16, dst16, zeros16)
    return _tc3(acc2, g2, degp, b2.reshape(1, 32), Wd, bd.reshape(1, 128))


# speedup vs baseline: 40.0654x; 1.0932x over previous
import functools

import jax
import jax.numpy as jnp
from jax import lax
from jax.experimental import pallas as pl
from jax.experimental.pallas import tpu as pltpu
from jax.experimental.pallas import tpu_sc as plsc

N = 10000
R = 10240
E = 320000
CHUNK = 125
NCHUNK = 80
NC2 = 160
RPT = R // 16
NB = 8
NG2 = NC2 // NB

_mesh = lambda: plsc.VectorSubcoreMesh(core_axis_name="c", subcore_axis_name="s")
_sc_params = lambda: pltpu.CompilerParams(use_tc_tiling_on_sc=False)


def _make_seg_half(dh):
    @functools.partial(
        pl.kernel,
        out_type=jax.ShapeDtypeStruct((2, R, dh), jnp.float32),
        mesh=_mesh(),
        scratch_types=[
            pltpu.VMEM((NC2, CHUNK), jnp.int32),
            pltpu.VMEM((NC2, CHUNK), jnp.int32),
            pltpu.VMEM((NB, CHUNK, dh), jnp.float32),
            pltpu.VMEM_SHARED((R, dh), jnp.float32),
            pltpu.VMEM_SHARED((R, dh), jnp.float32),
            pltpu.SemaphoreType.DMA((NB,)),
            pltpu.SemaphoreType.DMA((NB,)),
        ],
        compiler_params=_sc_params(),
    )
    def seg_half(g_hbm, src_hbm, dst_hbm, out_hbm,
                 src_v, dst_v, msg, acc, g_sp, sem_g, sem_s):
        c = lax.axis_index("c")
        s = lax.axis_index("s")
        row0 = s * RPT
        d0 = pltpu.async_copy(g_hbm.at[c, pl.ds(row0, RPT)],
                              acc.at[pl.ds(row0, RPT)], sem_g.at[0])
        d1 = pltpu.async_copy(g_hbm.at[c, pl.ds(row0, RPT)],
                              g_sp.at[pl.ds(row0, RPT)], sem_g.at[1])
        d2 = pltpu.async_copy(src_hbm.at[s], src_v, sem_g.at[2])
        d3 = pltpu.async_copy(dst_hbm.at[s], dst_v, sem_g.at[3])
        d0.wait(); d1.wait(); d2.wait(); d3.wait()
        plsc.subcore_barrier()

        def gather(j, b):
            pltpu.async_copy(g_sp.at[src_v.at[j]], msg.at[b], sem_g.at[b])

        def gather_wait(j, b):
            pltpu.make_async_copy(g_sp.at[src_v.at[j]], msg.at[b],
                                  sem_g.at[b]).wait()

        def scatter(j, b):
            pltpu.async_copy(msg.at[b], acc.at[dst_v.at[j]], sem_s.at[b],
                             add=True)

        def scatter_wait(j, b):
            pltpu.make_async_copy(msg.at[b], acc.at[dst_v.at[j]],
                                  sem_s.at[b]).wait()

        for b in range(NB):
            gather(b, b)

        def group(gi, carry):
            for b in range(NB):
                j = gi * NB + b
                gather_wait(j, b)
                scatter(j, b)
            for b in range(NB):
                j = gi * NB + b
                scatter_wait(j, b)
                gather(j + NB, b)
            return carry

        lax.fori_loop(0, NG2 - 1, group, 0)
        for b in range(NB):
            j = (NG2 - 1) * NB + b
            gather_wait(j, b)
            scatter(j, b)
        for b in range(NB):
            j = (NG2 - 1) * NB + b
            scatter_wait(j, b)

        plsc.subcore_barrier()
        pltpu.sync_copy(acc.at[pl.ds(row0, RPT)],
                        out_hbm.at[c, pl.ds(row0, RPT)])

    return seg_half


_seg_l1 = _make_seg_half(32)
_seg_l2 = _make_seg_half(16)


@functools.partial(
    pl.kernel,
    out_type=jax.ShapeDtypeStruct((2, R, 8), jnp.float32),
    mesh=_mesh(),
    scratch_types=[
        pltpu.VMEM((NCHUNK, CHUNK), jnp.int32),
        pltpu.VMEM((CHUNK, 8), jnp.float32),
        pltpu.VMEM_SHARED((R, 8), jnp.float32),
        pltpu.SemaphoreType.DMA((3,)),
    ],
    compiler_params=_sc_params(),
)
def _degree(dst_hbm, zeros_hbm, ones_hbm, out_hbm, dst_v, ones_v, acc, sem):
    c = lax.axis_index("c")
    s = lax.axis_index("s")
    w = c * 16 + s
    row0 = s * RPT
    d0 = pltpu.async_copy(zeros_hbm.at[pl.ds(row0, RPT)],
                          acc.at[pl.ds(row0, RPT)], sem.at[0])
    d1 = pltpu.async_copy(ones_hbm, ones_v, sem.at[1])
    d2 = pltpu.async_copy(dst_hbm.at[w], dst_v, sem.at[2])
    d0.wait(); d1.wait(); d2.wait()
    plsc.subcore_barrier()

    def fire(j, carry):
        pltpu.async_copy(ones_v, acc.at[dst_v.at[j]], sem.at[0], add=True)
        return carry

    lax.fori_loop(0, NCHUNK, fire, 0)

    def drain(j, carry):
        pltpu.make_async_copy(ones_v, acc.at[dst_v.at[j]], sem.at[0]).wait()
        return carry

    lax.fori_loop(0, NCHUNK, drain, 0)
    plsc.subcore_barrier()
    pltpu.sync_copy(acc.at[pl.ds(row0, RPT)], out_hbm.at[c, pl.ds(row0, RPT)])


def _dinv_of(dp_ref):
    deg = dp_ref[0, :, 0:1] + dp_ref[1, :, 0:1] + 1.0
    return lax.rsqrt(deg)


def _cat(a_ref):
    return jnp.concatenate([a_ref[0], a_ref[1]], axis=-1)


def _tc1_body(x_ref, w1_ref, dp_ref, g1_ref):
    t1 = jnp.dot(x_ref[...], w1_ref[...], preferred_element_type=jnp.float32)
    g1 = t1 * _dinv_of(dp_ref)
    g1_ref[0] = g1[:, :32]
    g1_ref[1] = g1[:, 32:]


def _tc2_body(a_ref, dp_ref, b1_ref, w2_ref, g2_ref):
    dinv = _dinv_of(dp_ref)
    h = jnp.maximum(dinv * _cat(a_ref) + b1_ref[...], 0.0)
    t2 = jnp.dot(h, w2_ref[...], preferred_element_type=jnp.float32)
    g2 = t2 * dinv
    g2_ref[0] = g2[:, :16]
    g2_ref[1] = g2[:, 16:]


def _tc3_body(a_ref, dp_ref, b2_ref, wd_ref, bd_ref, xh_ref):
    dinv = _dinv_of(dp_ref)
    z = dinv * _cat(a_ref) + b2_ref[...]
    xh_ref[...] = (jnp.dot(z, wd_ref[...], preferred_element_type=jnp.float32)
                   + bd_ref[...])


_BLK = 1280
_G = R // _BLK


def _tc1(x_pad, W1, degp):
    return pl.pallas_call(
        _tc1_body,
        grid=(_G,),
        in_specs=[
            pl.BlockSpec((_BLK, 128), lambda i: (i, 0)),
            pl.BlockSpec((128, 64), lambda i: (0, 0)),
            pl.BlockSpec((2, _BLK, 8), lambda i: (0, i, 0)),
        ],
        out_specs=pl.BlockSpec((2, _BLK, 32), lambda i: (0, i, 0)),
        out_shape=jax.ShapeDtypeStruct((2, R, 32), jnp.float32),
    )(x_pad, W1, degp)


def _tc2(acc1, degp, b1, W2):
    return pl.pallas_call(
        _tc2_body,
        grid=(_G,),
        in_specs=[
            pl.BlockSpec((2, _BLK, 32), lambda i: (0, i, 0)),
            pl.BlockSpec((2, _BLK, 8), lambda i: (0, i, 0)),
            pl.BlockSpec((1, 64), lambda i: (0, 0)),
            pl.BlockSpec((64, 32), lambda i: (0, 0)),
        ],
        out_specs=pl.BlockSpec((2, _BLK, 16), lambda i: (0, i, 0)),
        out_shape=jax.ShapeDtypeStruct((2, R, 16), jnp.float32),
    )(acc1, degp, b1, W2)


_BLK3 = 1000
_G3 = N // _BLK3


def _tc3(acc2, degp, b2, Wd, bd):
    return pl.pallas_call(
        _tc3_body,
        grid=(_G3,),
        in_specs=[
            pl.BlockSpec((2, _BLK3, 16), lambda i: (0, i, 0)),
            pl.BlockSpec((2, _BLK3, 8), lambda i: (0, i, 0)),
            pl.BlockSpec((1, 32), lambda i: (0, 0)),
            pl.BlockSpec((32, 128), lambda i: (0, 0)),
            pl.BlockSpec((1, 128), lambda i: (0, 0)),
        ],
        out_specs=pl.BlockSpec((_BLK3, 128), lambda i: (i, 0)),
        out_shape=jax.ShapeDtypeStruct((N, 128), jnp.float32),
    )(acc2, degp, b2, Wd, bd)


def kernel(x, edge_index, W1, b1, W2, b2, Wd, bd):
    src = edge_index[0].astype(jnp.int32)
    dst = edge_index[1].astype(jnp.int32)
    src16 = src.reshape(16, NC2, CHUNK)
    dst16 = dst.reshape(16, NC2, CHUNK)
    dst32 = dst.reshape(32, NCHUNK, CHUNK)
    x_pad = jnp.pad(x, ((0, R - N), (0, 0)))

    zeros8 = jnp.zeros((R, 8), jnp.float32)
    ones8 = jnp.ones((CHUNK, 8), jnp.float32)

    degp = _degree(dst32, zeros8, ones8)
    g1 = _tc1(x_pad, W1, degp)
    acc1 = _seg_l1(g1, src16, dst16)
    g2 = _tc2(acc1, degp, b1.reshape(1, 64), W2)
    acc2 = _seg_l2(g2, src16, dst16)
    return _tc3(acc2, degp, b2.reshape(1, 32), Wd, bd.reshape(1, 128))


# speedup vs baseline: 42.4699x; 1.0600x over previous
import functools

import jax
import jax.numpy as jnp
from jax import lax
from jax.experimental import pallas as pl
from jax.experimental.pallas import tpu as pltpu
from jax.experimental.pallas import tpu_sc as plsc

N = 10000
R = 10240
E = 320000
CHUNK = 125
NC2 = 160
NCHUNK = 80
RPT = R // 16
NB = 8
NG2 = NC2 // NB

_mesh = lambda: plsc.VectorSubcoreMesh(core_axis_name="c", subcore_axis_name="s")
_sc_params = lambda: pltpu.CompilerParams(use_tc_tiling_on_sc=False)


def _make_seg_half(dh):
    @functools.partial(
        pl.kernel,
        out_type=jax.ShapeDtypeStruct((2, R, dh), jnp.float32),
        mesh=_mesh(),
        scratch_types=[
            pltpu.VMEM((NC2, CHUNK), jnp.int32),
            pltpu.VMEM((NC2, CHUNK), jnp.int32),
            pltpu.VMEM((NB, CHUNK, dh), jnp.float32),
            pltpu.VMEM_SHARED((R, dh), jnp.float32),
            pltpu.VMEM_SHARED((R, dh), jnp.float32),
            pltpu.SemaphoreType.DMA((NB,)),
            pltpu.SemaphoreType.DMA((NB,)),
        ],
        compiler_params=_sc_params(),
    )
    def seg_half(eidx_hbm, g_hbm, out_hbm,
                 src_v, dst_v, msg, acc, g_sp, sem_g, sem_s):
        c = lax.axis_index("c")
        s = lax.axis_index("s")
        row0 = s * RPT
        d0 = pltpu.async_copy(g_hbm.at[c, pl.ds(row0, RPT)],
                              acc.at[pl.ds(row0, RPT)], sem_g.at[0])
        d1 = pltpu.async_copy(g_hbm.at[c, pl.ds(row0, RPT)],
                              g_sp.at[pl.ds(row0, RPT)], sem_g.at[1])
        d2 = pltpu.async_copy(eidx_hbm.at[0, s], src_v, sem_g.at[2])
        d3 = pltpu.async_copy(eidx_hbm.at[1, s], dst_v, sem_g.at[3])
        d0.wait(); d1.wait(); d2.wait(); d3.wait()
        plsc.subcore_barrier()

        def gather(j, b):
            pltpu.async_copy(g_sp.at[src_v.at[j]], msg.at[b], sem_g.at[b])

        def gather_wait(j, b):
            pltpu.make_async_copy(g_sp.at[src_v.at[j]], msg.at[b],
                                  sem_g.at[b]).wait()

        def scatter(j, b):
            pltpu.async_copy(msg.at[b], acc.at[dst_v.at[j]], sem_s.at[b],
                             add=True)

        def scatter_wait(j, b):
            pltpu.make_async_copy(msg.at[b], acc.at[dst_v.at[j]],
                                  sem_s.at[b]).wait()

        for b in range(NB):
            gather(b, b)

        def group(gi, carry):
            for b in range(NB):
                j = gi * NB + b
                gather_wait(j, b)
                scatter(j, b)
            for b in range(NB):
                j = gi * NB + b
                scatter_wait(j, b)
                gather(j + NB, b)
            return carry

        lax.fori_loop(0, NG2 - 1, group, 0)
        for b in range(NB):
            j = (NG2 - 1) * NB + b
            gather_wait(j, b)
            scatter(j, b)
        for b in range(NB):
            j = (NG2 - 1) * NB + b
            scatter_wait(j, b)

        plsc.subcore_barrier()
        pltpu.sync_copy(acc.at[pl.ds(row0, RPT)],
                        out_hbm.at[c, pl.ds(row0, RPT)])

    return seg_half


_seg_l1 = _make_seg_half(32)
_seg_l2 = _make_seg_half(16)


@functools.partial(
    pl.kernel,
    out_type=jax.ShapeDtypeStruct((2, R, 8), jnp.float32),
    mesh=_mesh(),
    scratch_types=[
        pltpu.VMEM((NCHUNK, CHUNK), jnp.int32),
        pltpu.VMEM((CHUNK, 8), jnp.float32),
        pltpu.VMEM_SHARED((R, 8), jnp.float32),
        pltpu.SemaphoreType.DMA((3,)),
    ],
    compiler_params=_sc_params(),
)
def _degree(eidx_hbm, zeros_hbm, ones_hbm, out_hbm, dst_v, ones_v, acc, sem):
    c = lax.axis_index("c")
    s = lax.axis_index("s")
    row0 = s * RPT
    d0 = pltpu.async_copy(zeros_hbm.at[pl.ds(row0, RPT)],
                          acc.at[pl.ds(row0, RPT)], sem.at[0])
    d1 = pltpu.async_copy(ones_hbm, ones_v, sem.at[1])
    d2 = pltpu.async_copy(eidx_hbm.at[1, s, pl.ds(c * NCHUNK, NCHUNK)],
                          dst_v, sem.at[2])
    d0.wait(); d1.wait(); d2.wait()
    plsc.subcore_barrier()

    def fire(j, carry):
        pltpu.async_copy(ones_v, acc.at[dst_v.at[j]], sem.at[0], add=True)
        return carry

    lax.fori_loop(0, NCHUNK, fire, 0)

    def drain(j, carry):
        pltpu.make_async_copy(ones_v, acc.at[dst_v.at[j]], sem.at[0]).wait()
        return carry

    lax.fori_loop(0, NCHUNK, drain, 0)
    plsc.subcore_barrier()
    pltpu.sync_copy(acc.at[pl.ds(row0, RPT)], out_hbm.at[c, pl.ds(row0, RPT)])


def _dinv_of(dp_ref):
    deg = dp_ref[0, :, 0:1] + dp_ref[1, :, 0:1] + 1.0
    return lax.rsqrt(deg)


def _cat(a_ref):
    return jnp.concatenate([a_ref[0], a_ref[1]], axis=-1)


def _tc1_body(x_ref, w1_ref, dp_ref, g1_ref):
    t1 = jnp.dot(x_ref[...], w1_ref[...], preferred_element_type=jnp.float32)
    g1 = t1 * _dinv_of(dp_ref)
    g1_ref[0] = g1[:, :32]
    g1_ref[1] = g1[:, 32:]


def _tc2_body(a_ref, dp_ref, b1_ref, w2_ref, g2_ref):
    dinv = _dinv_of(dp_ref)
    h = jnp.maximum(dinv * _cat(a_ref) + b1_ref[...], 0.0)
    t2 = jnp.dot(h, w2_ref[...], preferred_element_type=jnp.float32)
    g2 = t2 * dinv
    g2_ref[0] = g2[:, :16]
    g2_ref[1] = g2[:, 16:]


def _tc3_body(a_ref, dp_ref, b2_ref, wd_ref, bd_ref, xh_ref):
    dinv = _dinv_of(dp_ref)
    z = dinv * _cat(a_ref) + b2_ref[...]
    xh_ref[...] = (jnp.dot(z, wd_ref[...], preferred_element_type=jnp.float32)
                   + bd_ref[...])


_BLK = 2048
_G = R // _BLK


def _tc1(x_pad, W1, degp):
    return pl.pallas_call(
        _tc1_body,
        grid=(_G,),
        in_specs=[
            pl.BlockSpec((_BLK, 128), lambda i: (i, 0)),
            pl.BlockSpec((128, 64), lambda i: (0, 0)),
            pl.BlockSpec((2, _BLK, 8), lambda i: (0, i, 0)),
        ],
        out_specs=pl.BlockSpec((2, _BLK, 32), lambda i: (0, i, 0)),
        out_shape=jax.ShapeDtypeStruct((2, R, 32), jnp.float32),
    )(x_pad, W1, degp)


def _tc2(acc1, degp, b1, W2):
    return pl.pallas_call(
        _tc2_body,
        grid=(_G,),
        in_specs=[
            pl.BlockSpec((2, _BLK, 32), lambda i: (0, i, 0)),
            pl.BlockSpec((2, _BLK, 8), lambda i: (0, i, 0)),
            pl.BlockSpec((1, 64), lambda i: (0, 0)),
            pl.BlockSpec((64, 32), lambda i: (0, 0)),
        ],
        out_specs=pl.BlockSpec((2, _BLK, 16), lambda i: (0, i, 0)),
        out_shape=jax.ShapeDtypeStruct((2, R, 16), jnp.float32),
    )(acc1, degp, b1, W2)


def _tc3(acc2, degp, b2, Wd, bd):
    return pl.pallas_call(
        _tc3_body,
        grid=(_G,),
        in_specs=[
            pl.BlockSpec((2, _BLK, 16), lambda i: (0, i, 0)),
            pl.BlockSpec((2, _BLK, 8), lambda i: (0, i, 0)),
            pl.BlockSpec((1, 32), lambda i: (0, 0)),
            pl.BlockSpec((32, 128), lambda i: (0, 0)),
            pl.BlockSpec((1, 128), lambda i: (0, 0)),
        ],
        out_specs=pl.BlockSpec((_BLK, 128), lambda i: (i, 0)),
        out_shape=jax.ShapeDtypeStruct((N, 128), jnp.float32),
    )(acc2, degp, b2, Wd, bd)


def kernel(x, edge_index, W1, b1, W2, b2, Wd, bd):
    eidx = edge_index.astype(jnp.int32).reshape(2, 16, NC2, CHUNK)
    x_pad = jnp.pad(x, ((0, R - N), (0, 0)))

    zeros8 = jnp.zeros((R, 8), jnp.float32)
    ones8 = jnp.ones((CHUNK, 8), jnp.float32)

    degp = _degree(eidx, zeros8, ones8)
    g1 = _tc1(x_pad, W1, degp)
    acc1 = _seg_l1(eidx, g1)
    g2 = _tc2(acc1, degp, b1.reshape(1, 64), W2)
    acc2 = _seg_l2(eidx, g2)
    return _tc3(acc2, degp, b2.reshape(1, 32), Wd, bd.reshape(1, 128))


# speedup vs baseline: 51.2021x; 1.2056x over previous
import functools

import jax
import jax.numpy as jnp
from jax import lax
from jax.scipy.linalg import block_diag
from jax.experimental import pallas as pl
from jax.experimental.pallas import tpu as pltpu
from jax.experimental.pallas import tpu_sc as plsc

N = 10000
R = 10240
R4 = R // 4
E = 320000
CHUNK = 125
NC2 = 160
NCHUNK = 80
RPT = R // 16
NB = 8
NG2 = NC2 // NB
NG1 = NCHUNK // NB

_mesh = lambda: plsc.VectorSubcoreMesh(core_axis_name="c", subcore_axis_name="s")
_sc_params = lambda: pltpu.CompilerParams(use_tc_tiling_on_sc=False)


@functools.partial(
    pl.kernel,
    out_type=jax.ShapeDtypeStruct((2, R, 32), jnp.float32),
    mesh=_mesh(),
    scratch_types=[
        pltpu.VMEM((NC2, CHUNK), jnp.int32),
        pltpu.VMEM((NC2, CHUNK), jnp.int32),
        pltpu.VMEM((NB, CHUNK, 32), jnp.float32),
        pltpu.VMEM_SHARED((R, 32), jnp.float32),
        pltpu.VMEM_SHARED((R, 32), jnp.float32),
        pltpu.SemaphoreType.DMA((NB,)),
        pltpu.SemaphoreType.DMA((NB,)),
    ],
    compiler_params=_sc_params(),
)
def _seg_l1(eidx_hbm, g_hbm, out_hbm, src_v, dst_v, msg, acc, g_sp,
            sem_g, sem_s):
    c = lax.axis_index("c")
    s = lax.axis_index("s")
    row0 = s * RPT
    d0 = pltpu.async_copy(g_hbm.at[c, pl.ds(row0, RPT)],
                          acc.at[pl.ds(row0, RPT)], sem_g.at[0])
    d1 = pltpu.async_copy(g_hbm.at[c, pl.ds(row0, RPT)],
                          g_sp.at[pl.ds(row0, RPT)], sem_g.at[1])
    d2 = pltpu.async_copy(eidx_hbm.at[0, s], src_v, sem_g.at[2])
    d3 = pltpu.async_copy(eidx_hbm.at[1, s], dst_v, sem_g.at[3])
    d0.wait(); d1.wait(); d2.wait(); d3.wait()
    plsc.subcore_barrier()

    def gather(j, b):
        pltpu.async_copy(g_sp.at[src_v.at[j]], msg.at[b], sem_g.at[b])

    def gather_wait(j, b):
        pltpu.make_async_copy(g_sp.at[src_v.at[j]], msg.at[b],
                              sem_g.at[b]).wait()

    def scatter(j, b):
        pltpu.async_copy(msg.at[b], acc.at[dst_v.at[j]], sem_s.at[b],
                         add=True)

    def scatter_wait(j, b):
        pltpu.make_async_copy(msg.at[b], acc.at[dst_v.at[j]],
                              sem_s.at[b]).wait()

    for b in range(NB):
        gather(b, b)

    def group(gi, carry):
        for b in range(NB):
            j = gi * NB + b
            gather_wait(j, b)
            scatter(j, b)
        for b in range(NB):
            j = gi * NB + b
            scatter_wait(j, b)
            gather(j + NB, b)
        return carry

    lax.fori_loop(0, NG2 - 1, group, 0)
    for b in range(NB):
        j = (NG2 - 1) * NB + b
        gather_wait(j, b)
        scatter(j, b)
    for b in range(NB):
        j = (NG2 - 1) * NB + b
        scatter_wait(j, b)

    plsc.subcore_barrier()
    pltpu.sync_copy(acc.at[pl.ds(row0, RPT)],
                    out_hbm.at[c, pl.ds(row0, RPT)])


@functools.partial(
    pl.kernel,
    out_type=jax.ShapeDtypeStruct((2, R, 32), jnp.float32),
    mesh=_mesh(),
    scratch_types=[
        pltpu.VMEM((NCHUNK, CHUNK), jnp.int32),
        pltpu.VMEM((NCHUNK, CHUNK), jnp.int32),
        pltpu.VMEM((NB, CHUNK, 32), jnp.float32),
        pltpu.VMEM_SHARED((R, 32), jnp.float32),
        pltpu.VMEM_SHARED((R, 32), jnp.float32),
        pltpu.SemaphoreType.DMA((NB,)),
        pltpu.SemaphoreType.DMA((NB,)),
    ],
    compiler_params=_sc_params(),
)
def _seg_l2(eidx_hbm, g_hbm, out_hbm, src_v, dst_v, msg, acc, g_sp,
            sem_g, sem_s):
    c = lax.axis_index("c")
    s = lax.axis_index("s")
    row0 = s * RPT
    d0 = pltpu.async_copy(g_hbm.at[pl.ds(row0, RPT)],
                          g_sp.at[pl.ds(row0, RPT)], sem_g.at[0])
    d2 = pltpu.async_copy(eidx_hbm.at[0, s, pl.ds(c * NCHUNK, NCHUNK)],
                          src_v, sem_g.at[2])
    d3 = pltpu.async_copy(eidx_hbm.at[1, s, pl.ds(c * NCHUNK, NCHUNK)],
                          dst_v, sem_g.at[3])
    d0.wait(); d2.wait(); d3.wait()
    d1 = pltpu.async_copy(g_hbm.at[pl.ds(row0, RPT)],
                          acc.at[pl.ds(row0, RPT)], sem_g.at[1])
    d1.wait()
    plsc.subcore_barrier()

    def gather(j, b):
        pltpu.async_copy(g_sp.at[src_v.at[j]], msg.at[b], sem_g.at[b])

    def gather_wait(j, b):
        pltpu.make_async_copy(g_sp.at[src_v.at[j]], msg.at[b],
                              sem_g.at[b]).wait()

    def scatter(j, b):
        pltpu.async_copy(msg.at[b], acc.at[dst_v.at[j]], sem_s.at[b],
                         add=True)

    def scatter_wait(j, b):
        pltpu.make_async_copy(msg.at[b], acc.at[dst_v.at[j]],
                              sem_s.at[b]).wait()

    for b in range(NB):
        gather(b, b)

    def group(gi, carry):
        for b in range(NB):
            j = gi * NB + b
            gather_wait(j, b)
            scatter(j, b)
        for b in range(NB):
            j = gi * NB + b
            scatter_wait(j, b)
            gather(j + NB, b)
        return carry

    lax.fori_loop(0, NG1 - 1, group, 0)
    for b in range(NB):
        j = (NG1 - 1) * NB + b
        gather_wait(j, b)
        scatter(j, b)
    for b in range(NB):
        j = (NG1 - 1) * NB + b
        scatter_wait(j, b)

    plsc.subcore_barrier()
    pltpu.sync_copy(acc.at[pl.ds(row0, RPT)],
                    out_hbm.at[c, pl.ds(row0, RPT)])


@functools.partial(
    pl.kernel,
    out_type=jax.ShapeDtypeStruct((2, R, 32), jnp.float32),
    mesh=_mesh(),
    scratch_types=[
        pltpu.VMEM((NCHUNK, CHUNK), jnp.int32),
        pltpu.VMEM((CHUNK, 32), jnp.float32),
        pltpu.VMEM_SHARED((R, 32), jnp.float32),
        pltpu.SemaphoreType.DMA((3,)),
    ],
    compiler_params=_sc_params(),
)
def _degree(eidx_hbm, zeros_hbm, ones_hbm, out_hbm, dst_v, ones_v, acc, sem):
    c = lax.axis_index("c")
    s = lax.axis_index("s")
    row0 = s * RPT
    d0 = pltpu.async_copy(zeros_hbm.at[pl.ds(row0, RPT)],
                          acc.at[pl.ds(row0, RPT)], sem.at[0])
    d1 = pltpu.async_copy(ones_hbm, ones_v, sem.at[1])
    d2 = pltpu.async_copy(eidx_hbm.at[1, s, pl.ds(c * NCHUNK, NCHUNK)],
                          dst_v, sem.at[2])
    d0.wait(); d1.wait(); d2.wait()
    plsc.subcore_barrier()

    def fire(j, carry):
        pltpu.async_copy(ones_v, acc.at[dst_v.at[j]], sem.at[0], add=True)
        return carry

    lax.fori_loop(0, NCHUNK, fire, 0)

    def drain(j, carry):
        pltpu.make_async_copy(ones_v, acc.at[dst_v.at[j]], sem.at[0]).wait()
        return carry

    lax.fori_loop(0, NCHUNK, drain, 0)
    plsc.subcore_barrier()
    pltpu.sync_copy(acc.at[pl.ds(row0, RPT)], out_hbm.at[c, pl.ds(row0, RPT)])


_BLK4 = 512
_G = R4 // _BLK4


def _dinv4_of(dp_ref):
    deg = dp_ref[0] + dp_ref[1] + 1.0
    return lax.rsqrt(deg)


def _tc1_body(x_ref, w1a_ref, w1b_ref, dp_ref, g1_ref):
    dinv4 = _dinv4_of(dp_ref)
    g1_ref[0] = jnp.dot(x_ref[...], w1a_ref[...],
                        preferred_element_type=jnp.float32) * dinv4
    g1_ref[1] = jnp.dot(x_ref[...], w1b_ref[...],
                        preferred_element_type=jnp.float32) * dinv4


def _tc2_body(a_ref, dp_ref, b1a_ref, b1b_ref, w2a_ref, w2b_ref, g2_ref):
    dinv4 = _dinv4_of(dp_ref)
    ha = jnp.maximum(a_ref[0] * dinv4 + b1a_ref[...], 0.0)
    hb = jnp.maximum(a_ref[1] * dinv4 + b1b_ref[...], 0.0)
    t2 = (jnp.dot(ha, w2a_ref[...], preferred_element_type=jnp.float32)
          + jnp.dot(hb, w2b_ref[...], preferred_element_type=jnp.float32))
    g2_ref[...] = t2 * dinv4


def _tc3_body(a_ref, g2_ref, dp_ref, b2_ref, wd_ref, bd_ref, xh_ref):
    dinv4 = _dinv4_of(dp_ref)
    z = (a_ref[0] + a_ref[1] - g2_ref[...]) * dinv4 + b2_ref[...]
    cols = []
    for a in range(4):
        cols.append(jnp.dot(z, wd_ref[a], preferred_element_type=jnp.float32)
                    + bd_ref[...])
    xh_ref[...] = jnp.stack(cols, axis=1)


def _tc1(x_packed, W1a, W1b, degp):
    return pl.pallas_call(
        _tc1_body,
        grid=(_G,),
        in_specs=[
            pl.BlockSpec((_BLK4, 512), lambda i: (i, 0)),
            pl.BlockSpec((512, 128), lambda i: (0, 0)),
            pl.BlockSpec((512, 128), lambda i: (0, 0)),
            pl.BlockSpec((2, _BLK4, 128), lambda i: (0, i, 0)),
        ],
        out_specs=pl.BlockSpec((2, _BLK4, 128), lambda i: (0, i, 0)),
        out_shape=jax.ShapeDtypeStruct((2, R4, 128), jnp.float32),
    )(x_packed, W1a, W1b, degp)


def _tc2(acc1, degp, b1a, b1b, W2a, W2b):
    return pl.pallas_call(
        _tc2_body,
        grid=(_G,),
        in_specs=[
            pl.BlockSpec((2, _BLK4, 128), lambda i: (0, i, 0)),
            pl.BlockSpec((2, _BLK4, 128), lambda i: (0, i, 0)),
            pl.BlockSpec((1, 128), lambda i: (0, 0)),
            pl.BlockSpec((1, 128), lambda i: (0, 0)),
            pl.BlockSpec((128, 128), lambda i: (0, 0)),
            pl.BlockSpec((128, 128), lambda i: (0, 0)),
        ],
        out_specs=pl.BlockSpec((_BLK4, 128), lambda i: (i, 0)),
        out_shape=jax.ShapeDtypeStruct((R4, 128), jnp.float32),
    )(acc1, degp, b1a, b1b, W2a, W2b)


def _tc3(acc2, g2, degp, b2p, Wds, bd):
    return pl.pallas_call(
        _tc3_body,
        grid=(_G,),
        in_specs=[
            pl.BlockSpec((2, _BLK4, 128), lambda i: (0, i, 0)),
            pl.BlockSpec((_BLK4, 128), lambda i: (i, 0)),
            pl.BlockSpec((2, _BLK4, 128), lambda i: (0, i, 0)),
            pl.BlockSpec((1, 128), lambda i: (0, 0)),
            pl.BlockSpec((4, 128, 128), lambda i: (0, 0, 0)),
            pl.BlockSpec((1, 128), lambda i: (0, 0)),
        ],
        out_specs=pl.BlockSpec((_BLK4, 4, 128), lambda i: (i, 0, 0)),
        out_shape=jax.ShapeDtypeStruct((N // 4, 4, 128), jnp.float32),
    )(acc2, g2, degp, b2p, Wds, bd)


def kernel(x, edge_index, W1, b1, W2, b2, Wd, bd):
    eidx = edge_index.astype(jnp.int32).reshape(2, 16, NC2, CHUNK)
    x_packed = jnp.pad(x, ((0, R - N), (0, 0))).reshape(R4, 512)

    zeros32 = jnp.zeros((R, 32), jnp.float32)
    ones32 = jnp.ones((CHUNK, 32), jnp.float32)

    W1a = block_diag(*([W1[:, :32]] * 4))
    W1b = block_diag(*([W1[:, 32:]] * 4))
    W2a = block_diag(*([W2[:32, :]] * 4))
    W2b = block_diag(*([W2[32:, :]] * 4))
    Wds = jnp.stack([jnp.zeros((128, 128), W2.dtype)
                     .at[32 * a:32 * a + 32].set(Wd) for a in range(4)])
    b1a = jnp.tile(b1[:32], 4).reshape(1, 128)
    b1b = jnp.tile(b1[32:], 4).reshape(1, 128)
    b2p = jnp.tile(b2, 4).reshape(1, 128)
    bdp = bd.reshape(1, 128)

    degp = _degree(eidx, zeros32, ones32)
    degp4 = degp.reshape(2, R4, 128)
    g1 = _tc1(x_packed, W1a, W1b, degp4)
    acc1 = _seg_l1(eidx, g1.reshape(2, R, 32))
    g2 = _tc2(acc1.reshape(2, R4, 128), degp4, b1a, b1b, W2a, W2b)
    acc2 = _seg_l2(eidx, g2.reshape(R, 32))
    xh = _tc3(acc2.reshape(2, R4, 128), g2, degp4, b2p, Wds, bdp)
    return xh.reshape(N, 128)
